# baseline (speedup 1.0000x reference)
import functools

import jax
import jax.numpy as jnp
from jax import lax
from jax.experimental import pallas as pl
from jax.experimental.pallas import tpu as pltpu
from jax.experimental.pallas import tpu_sc as plsc

_STRIP = 8
_N_SC = 4
_ROWS = 16


def _tc_body(smooth_ref, inp_ref, tgt_ref, out_ref, r_ref):
    n = pl.program_id(0)
    num_n = pl.num_programs(0)
    smooth = smooth_ref[0, 0]
    C = inp_ref.shape[1]
    H = inp_ref.shape[2]
    W = inp_ref.shape[3]

    @pl.when(n == 0)
    def _init_r():
        r_ref[0] = 0.0

    one = jnp.float32(1.0)
    zero = jnp.float32(0.0)
    acc_i = [jnp.zeros((_STRIP, W), jnp.float32) for _ in range(C)]
    acc_d = [jnp.zeros((_STRIP, W), jnp.float32) for _ in range(C)]
    for s in range(0, H, _STRIP):
        tv = tgt_ref[0, pl.ds(s, _STRIP), :]
        for c in range(C):
            xv = inp_ref[0, c, pl.ds(s, _STRIP), :]
            eq = tv == c
            acc_i[c] = acc_i[c] + jnp.where(eq, xv, zero)
            acc_d[c] = acc_d[c] + (xv + jnp.where(eq, one, zero))

    r = jnp.float32(0.0)
    for c in range(C):
        inter = jnp.sum(acc_i[c])
        den = jnp.sum(acc_d[c])
        r = r + (2.0 * inter + smooth) / (den + smooth)
    r_ref[0] = r_ref[0] + r

    @pl.when(n == num_n - 1)
    def _fini():
        out_ref[0, 0] = r_ref[0]


def _tc_ratio_sum(inputs, targets, smooth_arr, n_tc):
    N, C, H, W = inputs.shape
    out = pl.pallas_call(
        _tc_body,
        grid=(n_tc,),
        in_specs=[
            pl.BlockSpec(memory_space=pltpu.SMEM),
            pl.BlockSpec((1, C, H, W), lambda n: (n, 0, 0, 0)),
            pl.BlockSpec((1, H, W), lambda n: (n, 0, 0)),
        ],
        out_specs=pl.BlockSpec(memory_space=pltpu.SMEM),
        out_shape=jax.ShapeDtypeStruct((1, 1), jnp.float32),
        scratch_shapes=[pltpu.SMEM((1,), jnp.float32)],
    )(smooth_arr, inputs, targets)
    return out[0, 0]


def _make_sc_kernel(N, C, H, W, n0):
    NC = 2
    NS = 16
    NW = NC * NS
    rows = H // NW
    mesh = plsc.VectorSubcoreMesh(core_axis_name="c", subcore_axis_name="s")
    n_sc = N - n0
    nvec = rows * W // 16

    @functools.partial(
        pl.kernel,
        mesh=mesh,
        out_type=jax.ShapeDtypeStruct((n_sc, NW, 2 * C, 16), jnp.float32),
        scratch_types=[
            pltpu.VMEM((2, C, rows, W), jnp.float32),
            pltpu.VMEM((2, rows, W), jnp.int32),
            pltpu.VMEM((2 * C, 16), jnp.float32),
            pltpu.SemaphoreType.DMA,
            pltpu.SemaphoreType.DMA,
        ],
    )
    def sc_dice(x_hbm, t_hbm, out_hbm, xbuf, tbuf, stage, sem_x, sem_t):
        wid = lax.axis_index("s") * NC + lax.axis_index("c")
        r0 = wid * rows

        def issue(k, b):
            hs = []
            for c in range(C):
                hs.append(pltpu.async_copy(
                    x_hbm.at[n0 + k, c, pl.ds(r0, rows), :],
                    xbuf.at[b, c], sem_x))
            hs.append(pltpu.async_copy(
                t_hbm.at[n0 + k, pl.ds(r0, rows), :],
                tbuf.at[b], sem_t))
            return hs

        pending = issue(0, 0)
        for k in range(n_sc):
            b = k % 2
            nxt = issue(k + 1, (k + 1) % 2) if k + 1 < n_sc else []
            for h in pending:
                h.wait()
            pending = nxt

            zero_v = jnp.zeros((16,), jnp.float32)
            init = tuple([zero_v] * (2 * C))

            def body(v, accs, _b=b):
                rr = v >> 5
                cc = (v & 31) * 16
                tv = tbuf[_b, rr, pl.ds(cc, 16)]
                out = []
                for c in range(C):
                    xv = xbuf[_b, c, rr, pl.ds(cc, 16)]
                    eq = tv == c
                    ai = accs[2 * c] + jnp.where(eq, xv, 0.0)
                    ad = accs[2 * c + 1] + (
                        xv + jnp.where(eq, jnp.float32(1.0), 0.0))
                    out.append(ai)
                    out.append(ad)
                return tuple(out)

            accs = lax.fori_loop(0, nvec, body, init, unroll=4)

            for q in range(2 * C):
                stage[q, :] = accs[q]
            pltpu.sync_copy(stage, out_hbm.at[k, wid])

    return sc_dice


def kernel(inputs, targets, smooth):
    N, C, H, W = inputs.shape
    n_sc = _N_SC
    n_tc = N - n_sc
    t32 = targets.astype(jnp.int32)
    smooth_f = jnp.asarray(smooth, jnp.float32)
    s_arr = smooth_f.reshape(1, 1)

    r_tc = _tc_ratio_sum(inputs, t32, s_arr, n_tc)

    sc_fn = _make_sc_kernel(N, C, H, W, n_tc)
    sc_part = sc_fn(inputs, t32)
    sums = jnp.sum(sc_part, axis=(1, 3))
    inter = sums[:, 0::2]
    den = sums[:, 1::2]
    r_sc = jnp.sum((2.0 * inter + smooth_f) / (den + smooth_f))

    return 1.0 - (r_tc + r_sc) / (N * C)


# speedup vs baseline: 1.5153x; 1.5153x over previous
import jax
import jax.numpy as jnp
from jax.experimental import pallas as pl
from jax.experimental.pallas import tpu as pltpu

_STRIP = 8


def _dice_body(smooth_ref, inp_ref, tgt_ref, out_ref, r_ref):
    n = pl.program_id(0)
    num_n = pl.num_programs(0)
    smooth = smooth_ref[0, 0]
    C = inp_ref.shape[1]
    H = inp_ref.shape[2]
    W = inp_ref.shape[3]

    @pl.when(n == 0)
    def _init_r():
        r_ref[0] = 0.0

    one = jnp.float32(1.0)
    zero = jnp.float32(0.0)
    r = jnp.float32(0.0)
    for c in range(C):
        acc_i = jnp.zeros((_STRIP, W), jnp.float32)
        acc_d = jnp.zeros((_STRIP, W), jnp.float32)
        for s in range(0, H, _STRIP):
            tv = tgt_ref[0, pl.ds(s, _STRIP), :]
            xv = inp_ref[0, c, pl.ds(s, _STRIP), :]
            m = jnp.where(tv == c, one, zero)
            acc_i = acc_i + m * xv
            acc_d = acc_d + (xv + m)
        inter = jnp.sum(acc_i)
        den = jnp.sum(acc_d)
        r = r + (2.0 * inter + smooth) / (den + smooth)
    r_ref[0] = r_ref[0] + r

    @pl.when(n == num_n - 1)
    def _fini():
        out_ref[0, 0] = 1.0 - r_ref[0] / (num_n * C)


def kernel(inputs, targets, smooth):
    N, C, H, W = inputs.shape
    t32 = targets.astype(jnp.int32)
    s = jnp.asarray(smooth, jnp.float32).reshape(1, 1)
    out = pl.pallas_call(
        _dice_body,
        grid=(N,),
        in_specs=[
            pl.BlockSpec(memory_space=pltpu.SMEM),
            pl.BlockSpec((1, C, H, W), lambda n: (n, 0, 0, 0)),
            pl.BlockSpec((1, H, W), lambda n: (n, 0, 0)),
        ],
        out_specs=pl.BlockSpec(memory_space=pltpu.SMEM),
        out_shape=jax.ShapeDtypeStruct((1, 1), jnp.float32),
        scratch_shapes=[pltpu.SMEM((1,), jnp.float32)],
    )(s, inputs, t32)
    return out[0, 0]


# speedup vs baseline: 1.5572x; 1.0277x over previous
import jax
import jax.numpy as jnp
from jax.experimental import pallas as pl
from jax.experimental.pallas import tpu as pltpu

_STRIP = 8


def _dice_body(smooth_ref, inp_ref, tgt_ref, out_ref, acc_i_ref, acc_d_ref):
    n = pl.program_id(0)
    num_n = pl.num_programs(0)
    smooth = smooth_ref[0, 0]
    C = inp_ref.shape[1]
    H = inp_ref.shape[2]
    W = inp_ref.shape[3]

    one = jnp.float32(1.0)
    zero = jnp.float32(0.0)
    for c in range(C):
        acc_i = jnp.zeros((_STRIP, W), jnp.float32)
        acc_d = jnp.zeros((_STRIP, W), jnp.float32)
        for s in range(0, H, _STRIP):
            tv = tgt_ref[0, pl.ds(s, _STRIP), :]
            xv = inp_ref[0, c, pl.ds(s, _STRIP), :]
            m = jnp.where(tv == c, one, zero)
            acc_i = acc_i + m * xv
            acc_d = acc_d + (xv + m)
        acc_i_ref[n, c, :] = jnp.sum(acc_i, axis=0)
        acc_d_ref[n, c, :] = jnp.sum(acc_d, axis=0)

    @pl.when(n == num_n - 1)
    def _fini():
        inter = jnp.sum(acc_i_ref[...], axis=-1)
        den = jnp.sum(acc_d_ref[...], axis=-1)
        ratios = (2.0 * inter + smooth) / (den + smooth)
        out_ref[0, 0] = 1.0 - jnp.sum(ratios) / (num_n * C)


def kernel(inputs, targets, smooth):
    N, C, H, W = inputs.shape
    t32 = targets.astype(jnp.int32)
    s = jnp.asarray(smooth, jnp.float32).reshape(1, 1)
    out = pl.pallas_call(
        _dice_body,
        grid=(N,),
        in_specs=[
            pl.BlockSpec(memory_space=pltpu.SMEM),
            pl.BlockSpec((1, C, H, W), lambda n: (n, 0, 0, 0)),
            pl.BlockSpec((1, H, W), lambda n: (n, 0, 0)),
        ],
        out_specs=pl.BlockSpec(memory_space=pltpu.SMEM),
        out_shape=jax.ShapeDtypeStruct((1, 1), jnp.float32),
        scratch_shapes=[
            pltpu.VMEM((N, C, W), jnp.float32),
            pltpu.VMEM((N, C, W), jnp.float32),
        ],
    )(s, inputs, t32)
    return out[0, 0]
